# Initial kernel scaffold; baseline (speedup 1.0000x reference)
import jax
import jax.numpy as jnp
from jax.experimental import pallas as pl


def kernel(x, embedding):
    raise NotImplementedError("write your pallas kernel here")


# speedup vs baseline: 2.5562x; 2.5562x over previous
import jax
import jax.numpy as jnp
from jax.experimental import pallas as pl

_NUM_ROWS = 2048
_DIM = 1024
_BATCH = 4
_BLOCK_ROWS = 256


def _body(x_ref, e_ref, o_ref):
    e = e_ref[0]
    r = pl.program_id(0)

    @pl.when(r == 0)
    def _():
        perm_head = jnp.concatenate(
            [e[0:1], e[3:4], e[6:7], e[1:2], e[4:5], e[7:8], e[2:3], e[5:6]],
            axis=0,
        )
        esum = jnp.concatenate([e[:8] + perm_head, 2.0 * e[8:]], axis=0)
        o_ref[...] = x_ref[...] + esum[None]

    @pl.when(r != 0)
    def _():
        o_ref[...] = x_ref[...] + 2.0 * e[None]


def kernel(x, embedding):
    grid = (_NUM_ROWS // _BLOCK_ROWS,)
    return pl.pallas_call(
        _body,
        grid=grid,
        in_specs=[
            pl.BlockSpec((_BATCH, _BLOCK_ROWS, _DIM), lambda r: (0, r, 0)),
            pl.BlockSpec((1, _BLOCK_ROWS, _DIM), lambda r: (0, r, 0)),
        ],
        out_specs=pl.BlockSpec((_BATCH, _BLOCK_ROWS, _DIM), lambda r: (0, r, 0)),
        out_shape=jax.ShapeDtypeStruct(x.shape, x.dtype),
    )(x, embedding)
